# baseline (speedup 1.0000x reference)
import functools

import jax
import jax.numpy as jnp
from jax import lax
from jax.experimental import pallas as pl
from jax.experimental.pallas import tpu as pltpu
from jax.experimental.pallas import tpu_sc as plsc

ENT_TOTAL = 100000
REL_TOTAL = 1000
HIDDEN = 128
LMBDA = 0.0001
BATCH = 16384

_info = plsc.get_sparse_core_info()
NC, NS, L = _info.num_cores, _info.num_subcores, _info.num_lanes
NW = NC * NS
B_PER_W = BATCH // NW
CHUNK = 32
NCHUNK = B_PER_W // CHUNK
FOLD = HIDDEN // L
TCROWS_PER_W = B_PER_W // FOLD
SQ_SLOTS = 6


def _sc_body(h_hbm, t_hbm, r_hbm, y_hbm, ent_hbm, rel_hbm,
             res_out, sq_out,
             ht_idx, idx_r, y_v,
             ht_rows, rows_r, resbuf, sqbuf, sem_i, sem_a, sem_b):
    wid = lax.axis_index("s") * NC + lax.axis_index("c")
    base = wid * B_PER_W

    idx_handles = []
    for c in range(NCHUNK):
        co = c * CHUNK
        idx_handles.append(pltpu.async_copy(
            h_hbm.at[pl.ds(base + co, CHUNK)],
            ht_idx.at[pl.ds(2 * co, CHUNK)], sem_i))
        idx_handles.append(pltpu.async_copy(
            t_hbm.at[pl.ds(base + co, CHUNK)],
            ht_idx.at[pl.ds(2 * co + CHUNK, CHUNK)], sem_i))
    idx_handles.append(pltpu.async_copy(
        r_hbm.at[pl.ds(base, B_PER_W)], idx_r, sem_i))
    idx_handles.append(pltpu.async_copy(
        y_hbm.at[pl.ds(base, B_PER_W)], y_v, sem_i))
    for hdl in idx_handles:
        hdl.wait()

    zero = jnp.zeros((L,), jnp.float32)

    sems = (sem_a, sem_b)

    def fire(c):
        p = c % 2
        co = c * CHUNK
        return (
            pltpu.async_copy(ent_hbm.at[ht_idx.at[pl.ds(2 * co, 2 * CHUNK)]],
                             ht_rows.at[p], sems[p]),
            pltpu.async_copy(rel_hbm.at[idx_r.at[pl.ds(co, CHUNK)]],
                             rows_r.at[p], sems[p]),
        )

    pending = fire(0)
    sq_acc = (zero,) * 6
    for c in range(NCHUNK):
        p = c % 2
        co = c * CHUNK
        handles = pending
        if c + 1 < NCHUNK:
            pending = fire(c + 1)
        for hdl in handles:
            hdl.wait()

        @plsc.parallel_loop(0, CHUNK, unroll=4, carry=sq_acc)
        def _row(i, carry):
            sh0, sh1, st0, st1, sr0, sr1 = carry
            gi = co + i
            yvec = y_v[pl.ds(jnp.bitwise_and(gi, ~(L - 1)), L)]
            ny = zero - jnp.take(
                yvec, jnp.full((L,), jnp.bitwise_and(gi, L - 1), jnp.int32))
            acc = jnp.zeros((L,), jnp.float32)
            for v in range(FOLD):
                sl = pl.ds(v * L, L)
                hv = ht_rows[p, i, sl]
                tv = ht_rows[p, CHUNK + i, sl]
                rv = rows_r[p, i, sl]
                acc = acc + hv * rv * tv
                if v % 2 == 0:
                    sh0 = sh0 + hv * hv
                    st0 = st0 + tv * tv
                    sr0 = sr0 + rv * rv
                else:
                    sh1 = sh1 + hv * hv
                    st1 = st1 + tv * tv
                    sr1 = sr1 + rv * rv
            tc_row = lax.shift_right_logical(gi, 3)
            tc_off = jnp.bitwise_and(gi, 7) * L
            resbuf[tc_row, pl.ds(tc_off, L)] = acc * ny
            return sh0, sh1, st0, st1, sr0, sr1

        sq_acc = _row

    for k, sq_part in enumerate(sq_acc):
        sqbuf[pl.ds(k * L, L)] = sq_part

    pltpu.sync_copy(resbuf, res_out.at[pl.ds(wid * TCROWS_PER_W, TCROWS_PER_W)])
    pltpu.sync_copy(sqbuf, sq_out.at[wid])


@functools.partial(
    pl.kernel,
    mesh=plsc.VectorSubcoreMesh(core_axis_name="c", subcore_axis_name="s"),
    out_type=[
        jax.ShapeDtypeStruct((BATCH // FOLD, HIDDEN), jnp.float32),
        jax.ShapeDtypeStruct((NW, SQ_SLOTS * L), jnp.float32),
    ],
    scratch_types=[
        pltpu.VMEM((2 * B_PER_W,), jnp.int32),
        pltpu.VMEM((B_PER_W,), jnp.int32),
        pltpu.VMEM((B_PER_W,), jnp.float32),
        pltpu.VMEM((2, 2 * CHUNK, HIDDEN), jnp.float32),
        pltpu.VMEM((2, CHUNK, HIDDEN), jnp.float32),
        pltpu.VMEM((TCROWS_PER_W, HIDDEN), jnp.float32),
        pltpu.VMEM((SQ_SLOTS * L,), jnp.float32),
        pltpu.SemaphoreType.DMA,
        pltpu.SemaphoreType.DMA,
        pltpu.SemaphoreType.DMA,
    ],
)
def _sc_gather_score(h_hbm, t_hbm, r_hbm, y_hbm, ent_hbm, rel_hbm,
                     res_out, sq_out,
                     ht_idx, idx_r, y_v, ht_rows, rows_r,
                     resbuf, sqbuf, sem_i, sem_a, sem_b):
    _sc_body(h_hbm, t_hbm, r_hbm, y_hbm, ent_hbm, rel_hbm, res_out, sq_out,
             ht_idx, idx_r, y_v, ht_rows, rows_r,
             resbuf, sqbuf, sem_i, sem_a, sem_b)


def _tc_body(rp_ref, sq_ref, out_ref):
    rp = rp_ref[...]
    d_ids = lax.broadcasted_iota(jnp.int32, (HIDDEN, FOLD), 0) // L
    j_ids = lax.broadcasted_iota(jnp.int32, (HIDDEN, FOLD), 1)
    seg = jnp.where(d_ids == j_ids, 1.0, 0.0).astype(jnp.float32)
    x = jnp.dot(rp, seg, preferred_element_type=jnp.float32)
    sp = jnp.log1p(jnp.exp(-jnp.abs(x))) + jnp.maximum(x, 0.0)
    loss = jnp.sum(sp) / BATCH
    reg = jnp.sum(sq_ref[...]) / (BATCH * HIDDEN)
    out_ref[...] = jnp.full((1, 1), loss + LMBDA * reg, jnp.float32)


def kernel(h, t, r, y, ent_embeddings, rel_embeddings):
    h = h.astype(jnp.int32)
    t = t.astype(jnp.int32)
    r = r.astype(jnp.int32)
    rp, sq = _sc_gather_score(h, t, r, y, ent_embeddings, rel_embeddings)
    loss = pl.pallas_call(
        _tc_body,
        out_shape=jax.ShapeDtypeStruct((1, 1), jnp.float32),
    )(rp, sq)
    return loss[0, 0]


# speedup vs baseline: 1.0956x; 1.0956x over previous
import functools

import jax
import jax.numpy as jnp
from jax import lax
from jax.experimental import pallas as pl
from jax.experimental.pallas import tpu as pltpu
from jax.experimental.pallas import tpu_sc as plsc

ENT_TOTAL = 100000
REL_TOTAL = 1000
HIDDEN = 128
LMBDA = 0.0001
BATCH = 16384

_info = plsc.get_sparse_core_info()
NC, NS, L = _info.num_cores, _info.num_subcores, _info.num_lanes
NW = NC * NS
B_PER_W = BATCH // NW
CHUNK = 64
NCHUNK = B_PER_W // CHUNK
FOLD = HIDDEN // L
TCROWS_PER_W = B_PER_W // FOLD
SQ_SLOTS = 6


def _sc_body(h_hbm, t_hbm, r_hbm, y_hbm, ent_hbm, rel_hbm,
             res_out, sq_out,
             ht_idx, idx_r, y_v,
             ht_rows, rows_r, resbuf, sqbuf, sem_i, sem_a, sem_b):
    wid = lax.axis_index("s") * NC + lax.axis_index("c")
    base = wid * B_PER_W

    idx_handles = []
    for c in range(NCHUNK):
        co = c * CHUNK
        idx_handles.append(pltpu.async_copy(
            h_hbm.at[pl.ds(base + co, CHUNK)],
            ht_idx.at[pl.ds(2 * co, CHUNK)], sem_i))
        idx_handles.append(pltpu.async_copy(
            t_hbm.at[pl.ds(base + co, CHUNK)],
            ht_idx.at[pl.ds(2 * co + CHUNK, CHUNK)], sem_i))
    idx_handles.append(pltpu.async_copy(
        r_hbm.at[pl.ds(base, B_PER_W)], idx_r, sem_i))
    idx_handles.append(pltpu.async_copy(
        y_hbm.at[pl.ds(base, B_PER_W)], y_v, sem_i))
    for hdl in idx_handles:
        hdl.wait()

    zero = jnp.zeros((L,), jnp.float32)

    sems = (sem_a, sem_b)

    def fire(c):
        p = c % 2
        co = c * CHUNK
        return (
            pltpu.async_copy(ent_hbm.at[ht_idx.at[pl.ds(2 * co, 2 * CHUNK)]],
                             ht_rows.at[p], sems[p]),
            pltpu.async_copy(rel_hbm.at[idx_r.at[pl.ds(co, CHUNK)]],
                             rows_r.at[p], sems[p]),
        )

    pending = fire(0)
    sq_acc = (zero,) * 6
    for c in range(NCHUNK):
        p = c % 2
        co = c * CHUNK
        handles = pending
        if c + 1 < NCHUNK:
            pending = fire(c + 1)
        for hdl in handles:
            hdl.wait()

        @plsc.parallel_loop(0, CHUNK, unroll=4, carry=sq_acc)
        def _row(i, carry):
            sh0, sh1, st0, st1, sr0, sr1 = carry
            gi = co + i
            yvec = y_v[pl.ds(jnp.bitwise_and(gi, ~(L - 1)), L)]
            ny = zero - jnp.take(
                yvec, jnp.full((L,), jnp.bitwise_and(gi, L - 1), jnp.int32))
            acc = jnp.zeros((L,), jnp.float32)
            for v in range(FOLD):
                sl = pl.ds(v * L, L)
                hv = ht_rows[p, i, sl]
                tv = ht_rows[p, CHUNK + i, sl]
                rv = rows_r[p, i, sl]
                acc = acc + hv * rv * tv
                if v % 2 == 0:
                    sh0 = sh0 + hv * hv
                    st0 = st0 + tv * tv
                    sr0 = sr0 + rv * rv
                else:
                    sh1 = sh1 + hv * hv
                    st1 = st1 + tv * tv
                    sr1 = sr1 + rv * rv
            tc_row = lax.shift_right_logical(gi, 3)
            tc_off = jnp.bitwise_and(gi, 7) * L
            resbuf[tc_row, pl.ds(tc_off, L)] = acc * ny
            return sh0, sh1, st0, st1, sr0, sr1

        sq_acc = _row

    for k, sq_part in enumerate(sq_acc):
        sqbuf[pl.ds(k * L, L)] = sq_part

    pltpu.sync_copy(resbuf, res_out.at[pl.ds(wid * TCROWS_PER_W, TCROWS_PER_W)])
    pltpu.sync_copy(sqbuf, sq_out.at[wid])


@functools.partial(
    pl.kernel,
    mesh=plsc.VectorSubcoreMesh(core_axis_name="c", subcore_axis_name="s"),
    out_type=[
        jax.ShapeDtypeStruct((BATCH // FOLD, HIDDEN), jnp.float32),
        jax.ShapeDtypeStruct((NW, SQ_SLOTS * L), jnp.float32),
    ],
    scratch_types=[
        pltpu.VMEM((2 * B_PER_W,), jnp.int32),
        pltpu.VMEM((B_PER_W,), jnp.int32),
        pltpu.VMEM((B_PER_W,), jnp.float32),
        pltpu.VMEM((2, 2 * CHUNK, HIDDEN), jnp.float32),
        pltpu.VMEM((2, CHUNK, HIDDEN), jnp.float32),
        pltpu.VMEM((TCROWS_PER_W, HIDDEN), jnp.float32),
        pltpu.VMEM((SQ_SLOTS * L,), jnp.float32),
        pltpu.SemaphoreType.DMA,
        pltpu.SemaphoreType.DMA,
        pltpu.SemaphoreType.DMA,
    ],
)
def _sc_gather_score(h_hbm, t_hbm, r_hbm, y_hbm, ent_hbm, rel_hbm,
                     res_out, sq_out,
                     ht_idx, idx_r, y_v, ht_rows, rows_r,
                     resbuf, sqbuf, sem_i, sem_a, sem_b):
    _sc_body(h_hbm, t_hbm, r_hbm, y_hbm, ent_hbm, rel_hbm, res_out, sq_out,
             ht_idx, idx_r, y_v, ht_rows, rows_r,
             resbuf, sqbuf, sem_i, sem_a, sem_b)


def _tc_body(rp_ref, sq_ref, out_ref):
    rp = rp_ref[...]
    d_ids = lax.broadcasted_iota(jnp.int32, (HIDDEN, FOLD), 0) // L
    j_ids = lax.broadcasted_iota(jnp.int32, (HIDDEN, FOLD), 1)
    seg = jnp.where(d_ids == j_ids, 1.0, 0.0).astype(jnp.float32)
    x = jnp.dot(rp, seg, preferred_element_type=jnp.float32)
    sp = jnp.log1p(jnp.exp(-jnp.abs(x))) + jnp.maximum(x, 0.0)
    loss = jnp.sum(sp) / BATCH
    reg = jnp.sum(sq_ref[...]) / (BATCH * HIDDEN)
    out_ref[...] = jnp.full((1, 1), loss + LMBDA * reg, jnp.float32)


def kernel(h, t, r, y, ent_embeddings, rel_embeddings):
    h = h.astype(jnp.int32)
    t = t.astype(jnp.int32)
    r = r.astype(jnp.int32)
    rp, sq = _sc_gather_score(h, t, r, y, ent_embeddings, rel_embeddings)
    loss = pl.pallas_call(
        _tc_body,
        out_shape=jax.ShapeDtypeStruct((1, 1), jnp.float32),
    )(rp, sq)
    return loss[0, 0]
